# baseline (speedup 1.0000x reference)
import functools

import jax
import jax.numpy as jnp
from jax import lax
from jax.experimental import pallas as pl
from jax.experimental.pallas import tpu as pltpu
from jax.experimental.pallas import tpu_sc as plsc

N = 10000
E = 320000
H = 128
NB = 32
NM = 250
MS = 40
FD = 200
FH = 512
NT = 12

NC = 2
NS = 16
NW = NC * NS

NPAD = 10240
APW = NPAD // NW
IPC = 128
APC = IPC // NB
NCH = APW * NB // IPC
IPW = APW * NB
REAL_LAST = N * NB - (NW - 1) * IPW
NBUF = 4

ROWBLK = 2000
MPB = ROWBLK // MS


PRE = NBUF * IPC
WPART = APW // 4


def _sc_gather_sum_body(bond_hbm, a2b_hbm, out_hbm,
                        idx_v, rows_bufs, acc_v, sems, sem_w):
    w = lax.axis_index("s") * NC + lax.axis_index("c")
    last = NW - 1

    def fire(ci, b):
        pltpu.async_copy(bond_hbm.at[idx_v.at[pl.ds(ci * IPC, IPC)]],
                         rows_bufs[b], sems[b])

    def wait(b):
        pltpu.make_async_copy(bond_hbm.at[idx_v.at[pl.ds(0, IPC)]],
                              rows_bufs[b], sems[b]).wait()

    pltpu.sync_copy(a2b_hbm.at[pl.ds(w * IPW, PRE)], idx_v.at[pl.ds(0, PRE)])
    for b in range(NBUF):
        fire(b, b)

    @pl.when(w < last)
    def _():
        pltpu.sync_copy(a2b_hbm.at[pl.ds(w * IPW + PRE, IPW - PRE)],
                        idx_v.at[pl.ds(PRE, IPW - PRE)])

    @pl.when(w == last)
    def _():
        pltpu.sync_copy(a2b_hbm.at[pl.ds(w * IPW + PRE, REAL_LAST - PRE)],
                        idx_v.at[pl.ds(PRE, REAL_LAST - PRE)])

        def fill(i, _):
            idx_v[pl.ds(REAL_LAST + 16 * i, 16)] = (
                lax.iota(jnp.int32, 16) + 16 * i)
            return 0
        lax.fori_loop(0, (IPW - REAL_LAST) // 16, fill, 0)

    def reduce_chunk(b, ci):
        rows = rows_bufs[b]
        for a in range(APC):
            base = a * NB

            def rbody(r, carry):
                r0 = base + r * 4
                out = carry
                for rr in range(4):
                    out = tuple(out[v] + rows[r0 + rr, pl.ds(v * 16, 16)]
                                for v in range(8))
                return out

            init = tuple(jnp.zeros((16,), jnp.float32) for _ in range(8))
            accs = lax.fori_loop(0, NB // 4, rbody, init)
            arow = APC * ci + a
            for v in range(4):
                packed = plsc.pack(accs[2 * v], accs[2 * v + 1],
                                   format=plsc.PackFormat.INTERLEAVED)
                acc_v[pl.ds(arow * (H // 2) + v * 16, 16)] = plsc.bitcast(
                    packed, jnp.int32)

    HW = H // 2

    def write_part(p):
        pltpu.async_copy(acc_v.at[pl.ds(p * WPART * HW, WPART * HW)],
                         out_hbm.at[pl.ds((w * APW + p * WPART) * HW,
                                          WPART * HW)],
                         sem_w)

    def outer(t, _):
        ci = NBUF * t
        for b in range(NBUF):
            wait(b)
            reduce_chunk(b, ci + b)
            fire(ci + b + NBUF, b)
        for p in range(3):
            @pl.when(t == (p + 1) * (WPART // 16) - 1)
            def _():
                write_part(p)
        return 0

    lax.fori_loop(0, NCH // NBUF - 1, outer, 0)
    for b in range(NBUF):
        wait(b)
        reduce_chunk(b, NCH - NBUF + b)
    write_part(3)
    for _ in range(4):
        pltpu.make_async_copy(acc_v.at[pl.ds(0, WPART * HW)],
                              out_hbm.at[pl.ds(w * APW * HW, WPART * HW)],
                              sem_w).wait()


@functools.cache
def _sc_gather_sum():
    return pl.kernel(
        _sc_gather_sum_body,
        out_type=jax.ShapeDtypeStruct((NPAD * H // 2,), jnp.int32),
        mesh=plsc.VectorSubcoreMesh(core_axis_name="c", subcore_axis_name="s",
                                    num_cores=NC, num_subcores=NS),
        scratch_types=[
            pltpu.VMEM((IPW,), jnp.int32),
            [pltpu.VMEM((IPC, H), jnp.float32) for _ in range(NBUF)],
            pltpu.VMEM((APW * H // 2,), jnp.int32),
            [pltpu.SemaphoreType.DMA for _ in range(NBUF)],
            pltpu.SemaphoreType.DMA,
        ],
        compiler_params=pltpu.CompilerParams(needs_layout_passes=False),
    )


def _tc_dense_body(of_ref, ag_ref, w1a_ref, w1b_ref, b1_ref, w2_ref, b2_ref,
                   g_ref, bb_ref, feat_ref, wf1a_ref, wf1b_ref, bf1_ref,
                   wf2_ref, bf2_ref, out_ref):
    f32 = jnp.float32
    bf = jnp.bfloat16
    mm = functools.partial(jnp.dot, preferred_element_type=f32)
    h = (mm(of_ref[...], w1a_ref[...])
         + mm(ag_ref[...], w1b_ref[...]) + b1_ref[...])
    h = jnp.maximum(h, 0.0)
    y = mm(h.astype(bf), w2_ref[...]) + b2_ref[...]
    mu = jnp.mean(y, axis=1, keepdims=True)
    var = jnp.mean((y - mu) ** 2, axis=1, keepdims=True)
    z = (y - mu) * lax.rsqrt(var + 1e-6) * g_ref[...] + bb_ref[...]
    rows = lax.broadcasted_iota(jnp.int32, (MPB, ROWBLK), 0)
    cols = lax.broadcasted_iota(jnp.int32, (MPB, ROWBLK), 1)
    sel = jnp.where(cols // MS == rows, f32(1.0 / MS), f32(0.0))
    zm = sel @ z
    m1 = zm @ wf1a_ref[...] + feat_ref[0] @ wf1b_ref[...] + bf1_ref[...]
    m1 = jnp.maximum(m1, 0.0)
    out_ref[0] = (m1 @ wf2_ref[...] + bf2_ref[...]) * 0.5


def _tc_dense(of, ag, w1a, w1b, b1, w2, b2, g, bb, feat, wf1a, wf1b, bf1,
              wf2, bf2):
    grid = N // ROWBLK
    full = lambda r, c: pl.BlockSpec((r, c), lambda i: (0, 0))
    return pl.pallas_call(
        _tc_dense_body,
        grid=(grid,),
        in_specs=[
            pl.BlockSpec((ROWBLK, H), lambda i: (i, 0)),
            pl.BlockSpec((ROWBLK, H), lambda i: (i, 0)),
            full(H, 4 * H),
            full(H, 4 * H),
            full(1, 4 * H),
            full(4 * H, H),
            full(1, H),
            full(1, H),
            full(1, H),
            pl.BlockSpec((1, MPB, FD), lambda i: (i, 0, 0)),
            full(H, FH),
            full(FD, FH),
            full(1, FH),
            full(FH, NT),
            full(1, NT),
        ],
        out_specs=pl.BlockSpec((1, MPB, NT), lambda i: (i, 0, 0)),
        out_shape=jax.ShapeDtypeStruct((grid, MPB, NT), jnp.float32),
    )(of, ag, w1a, w1b, b1, w2, b2, g, bb,
      feat.reshape(grid, MPB, FD), wf1a, wf1b, bf1, wf2, bf2
      ).reshape(NM, NT)


def kernel(atom_output, bond_output, original_f_atoms, original_f_bonds,
           a2a, a2b, b2a, b2revb, a_scope, features_batch,
           W_aa1, b_aa1, W_aa2, b_aa2, ln_aa_g, ln_aa_b,
           W_ab1, b_ab1, W_ab2, b_ab2, ln_ab_g, ln_ab_b,
           W_f1, b_f1, W_f2, b_f2):
    bf = jnp.bfloat16
    idx = a2b.astype(jnp.int32).reshape(-1)
    aggr = _sc_gather_sum()(bond_output, idx)
    aggr = lax.bitcast_convert_type(aggr, jnp.bfloat16).reshape(NPAD, H)

    s = jnp.arange(H)
    perm = (s // 32) * 32 + (s % 32) // 2 + 16 * (s % 2)
    out = _tc_dense(
        original_f_atoms.astype(bf), aggr,
        W_ab1[:H].astype(bf), W_ab1[H:][perm].astype(bf),
        b_ab1.reshape(1, 4 * H),
        W_ab2.astype(bf), b_ab2.reshape(1, H),
        ln_ab_g.reshape(1, H), ln_ab_b.reshape(1, H),
        features_batch,
        W_f1[:H], W_f1[H:], b_f1.reshape(1, FH),
        W_f2, b_f2.reshape(1, NT),
    )
    return out


# speedup vs baseline: 1.2630x; 1.2630x over previous
import functools

import jax
import jax.numpy as jnp
from jax import lax
from jax.experimental import pallas as pl
from jax.experimental.pallas import tpu as pltpu
from jax.experimental.pallas import tpu_sc as plsc

N = 10000
E = 320000
H = 128
NB = 32
NM = 250
MS = 40
FD = 200
FH = 512
NT = 12

NC = 2
NS = 16
NW = NC * NS

NPAD = 10240
APW = NPAD // NW
IPC = 128
APC = IPC // NB
NCH = APW * NB // IPC
IPW = APW * NB
REAL_LAST = N * NB - (NW - 1) * IPW
NBUF = 4

ROWBLK = 2000
MPB = ROWBLK // MS


PRE = NBUF * IPC
WPART = APW // 4


def _sc_gather_sum_body(bond_hbm, a2b_hbm, out_hbm,
                        idx_v, rows_bufs, acc_v, sems, sem_w):
    w = lax.axis_index("s") * NC + lax.axis_index("c")
    last = NW - 1

    def fire(ci, b):
        pltpu.async_copy(bond_hbm.at[idx_v.at[pl.ds(ci * IPC, IPC)]],
                         rows_bufs[b], sems[b])

    def wait(b):
        pltpu.make_async_copy(bond_hbm.at[idx_v.at[pl.ds(0, IPC)]],
                              rows_bufs[b], sems[b]).wait()

    pltpu.sync_copy(a2b_hbm.at[pl.ds(w * IPW, PRE)], idx_v.at[pl.ds(0, PRE)])
    for b in range(NBUF):
        fire(b, b)

    @pl.when(w < last)
    def _():
        pltpu.sync_copy(a2b_hbm.at[pl.ds(w * IPW + PRE, IPW - PRE)],
                        idx_v.at[pl.ds(PRE, IPW - PRE)])

    @pl.when(w == last)
    def _():
        pltpu.sync_copy(a2b_hbm.at[pl.ds(w * IPW + PRE, REAL_LAST - PRE)],
                        idx_v.at[pl.ds(PRE, REAL_LAST - PRE)])

        def fill(i, _):
            idx_v[pl.ds(REAL_LAST + 16 * i, 16)] = (
                lax.iota(jnp.int32, 16) + 16 * i)
            return 0
        lax.fori_loop(0, (IPW - REAL_LAST) // 16, fill, 0)

    def reduce_chunk(b, ci):
        rows = rows_bufs[b]
        for a in range(APC):
            base = a * NB

            def rbody(r, carry):
                r0 = base + r * 4
                out = carry
                for rr in range(4):
                    out = tuple(out[v] + rows[r0 + rr, pl.ds(v * 16, 16)]
                                for v in range(8))
                return out

            init = tuple(jnp.zeros((16,), jnp.float32) for _ in range(8))
            accs = lax.fori_loop(0, NB // 4, rbody, init)
            arow = APC * ci + a
            for v in range(8):
                acc_v[pl.ds(arow * H + v * 16, 16)] = accs[v]

    def write_part(p):
        pltpu.async_copy(acc_v.at[pl.ds(p * WPART * H, WPART * H)],
                         out_hbm.at[pl.ds((w * APW + p * WPART) * H,
                                          WPART * H)],
                         sem_w)

    def outer(t, _):
        ci = NBUF * t
        for b in range(NBUF):
            wait(b)
            reduce_chunk(b, ci + b)
            fire(ci + b + NBUF, b)
        for p in range(3):
            @pl.when(t == (p + 1) * (WPART // 16) - 1)
            def _():
                write_part(p)
        return 0

    lax.fori_loop(0, NCH // NBUF - 1, outer, 0)
    for b in range(NBUF):
        wait(b)
        reduce_chunk(b, NCH - NBUF + b)
    write_part(3)
    for _ in range(4):
        pltpu.make_async_copy(acc_v.at[pl.ds(0, WPART * H)],
                              out_hbm.at[pl.ds(w * APW * H, WPART * H)],
                              sem_w).wait()


@functools.cache
def _sc_gather_sum():
    return pl.kernel(
        _sc_gather_sum_body,
        out_type=jax.ShapeDtypeStruct((NPAD * H,), jnp.float32),
        mesh=plsc.VectorSubcoreMesh(core_axis_name="c", subcore_axis_name="s",
                                    num_cores=NC, num_subcores=NS),
        scratch_types=[
            pltpu.VMEM((IPW,), jnp.int32),
            [pltpu.VMEM((IPC, H), jnp.float32) for _ in range(NBUF)],
            pltpu.VMEM((APW * H,), jnp.float32),
            [pltpu.SemaphoreType.DMA for _ in range(NBUF)],
            pltpu.SemaphoreType.DMA,
        ],
    )


def _tc_dense_body(of_ref, ag_ref, w1a_ref, w1b_ref, b1_ref, w2_ref, b2_ref,
                   g_ref, bb_ref, feat_ref, wf1a_ref, wf1b_ref, bf1_ref,
                   wf2_ref, bf2_ref, out_ref):
    f32 = jnp.float32
    bf = jnp.bfloat16
    mm = functools.partial(jnp.dot, preferred_element_type=f32)
    h = (mm(of_ref[...], w1a_ref[...])
         + mm(ag_ref[...].astype(bf), w1b_ref[...]) + b1_ref[...])
    h = jnp.maximum(h, 0.0)
    y = mm(h.astype(bf), w2_ref[...]) + b2_ref[...]
    mu = jnp.mean(y, axis=1, keepdims=True)
    var = jnp.mean((y - mu) ** 2, axis=1, keepdims=True)
    z = (y - mu) * lax.rsqrt(var + 1e-6) * g_ref[...] + bb_ref[...]
    rows = lax.broadcasted_iota(jnp.int32, (MPB, ROWBLK), 0)
    cols = lax.broadcasted_iota(jnp.int32, (MPB, ROWBLK), 1)
    sel = jnp.where(cols // MS == rows, f32(1.0 / MS), f32(0.0))
    zm = sel @ z
    m1 = zm @ wf1a_ref[...] + feat_ref[0] @ wf1b_ref[...] + bf1_ref[...]
    m1 = jnp.maximum(m1, 0.0)
    out_ref[0] = (m1 @ wf2_ref[...] + bf2_ref[...]) * 0.5


def _tc_dense(of, ag, w1a, w1b, b1, w2, b2, g, bb, feat, wf1a, wf1b, bf1,
              wf2, bf2):
    grid = N // ROWBLK
    full = lambda r, c: pl.BlockSpec((r, c), lambda i: (0, 0))
    return pl.pallas_call(
        _tc_dense_body,
        grid=(grid,),
        in_specs=[
            pl.BlockSpec((ROWBLK, H), lambda i: (i, 0)),
            pl.BlockSpec((ROWBLK, H), lambda i: (i, 0)),
            full(H, 4 * H),
            full(H, 4 * H),
            full(1, 4 * H),
            full(4 * H, H),
            full(1, H),
            full(1, H),
            full(1, H),
            pl.BlockSpec((1, MPB, FD), lambda i: (i, 0, 0)),
            full(H, FH),
            full(FD, FH),
            full(1, FH),
            full(FH, NT),
            full(1, NT),
        ],
        out_specs=pl.BlockSpec((1, MPB, NT), lambda i: (i, 0, 0)),
        out_shape=jax.ShapeDtypeStruct((grid, MPB, NT), jnp.float32),
    )(of, ag, w1a, w1b, b1, w2, b2, g, bb,
      feat.reshape(grid, MPB, FD), wf1a, wf1b, bf1, wf2, bf2
      ).reshape(NM, NT)


def kernel(atom_output, bond_output, original_f_atoms, original_f_bonds,
           a2a, a2b, b2a, b2revb, a_scope, features_batch,
           W_aa1, b_aa1, W_aa2, b_aa2, ln_aa_g, ln_aa_b,
           W_ab1, b_ab1, W_ab2, b_ab2, ln_ab_g, ln_ab_b,
           W_f1, b_f1, W_f2, b_f2):
    bf = jnp.bfloat16
    idx = a2b.astype(jnp.int32).reshape(-1)
    aggr = _sc_gather_sum()(bond_output, idx).reshape(NPAD, H)

    out = _tc_dense(
        original_f_atoms.astype(bf), aggr,
        W_ab1[:H].astype(bf), W_ab1[H:].astype(bf),
        b_ab1.reshape(1, 4 * H),
        W_ab2.astype(bf), b_ab2.reshape(1, H),
        ln_ab_g.reshape(1, H), ln_ab_b.reshape(1, H),
        features_batch,
        W_f1[:H], W_f1[H:], b_f1.reshape(1, FH),
        W_f2, b_f2.reshape(1, NT),
    )
    return out
